# baseline (speedup 1.0000x reference)
import jax
import jax.numpy as jnp
from jax.experimental import pallas as pl
from jax.experimental.pallas import tpu as pltpu

B = 4096
I = 4096
O = 4096
CH = 64
NW = O // CH
NX = B // CH
NBUF = 32


def _body(x_hbm, w_hbm, b_ref, o_ref, buf, ws_ref, sems):
    def copy(i):
        if i < NW:
            src = w_hbm.at[pl.ds(i * CH, CH), :]
        else:
            src = x_hbm.at[pl.ds((i - NW) * CH, CH), :]
        return pltpu.make_async_copy(src, buf.at[i % NBUF], sems.at[i % NBUF])

    for i in range(NBUF):
        copy(i).start()

    bsum = jnp.sum(b_ref[...])

    for i in range(NW + NX):
        copy(i).wait()
        data = buf[i % NBUF]
        if i == 0:
            ws_ref[...] = jnp.sum(data, axis=0, keepdims=True)
        elif i < NW:
            ws_ref[...] += jnp.sum(data, axis=0, keepdims=True)
        else:
            part = jnp.sum(data * ws_ref[...], axis=1, keepdims=True)
            o_ref[pl.ds((i - NW) * CH, CH), :] = part + bsum
        if i + NBUF < NW + NX:
            copy(i + NBUF).start()


def kernel(x, W, b):
    return pl.pallas_call(
        _body,
        in_specs=[
            pl.BlockSpec(memory_space=pltpu.MemorySpace.HBM),
            pl.BlockSpec(memory_space=pltpu.MemorySpace.HBM),
            pl.BlockSpec((1, I), lambda: (0, 0)),
        ],
        out_specs=pl.BlockSpec((B, 1), lambda: (0, 0)),
        out_shape=jax.ShapeDtypeStruct((B, 1), jnp.float32),
        scratch_shapes=[
            pltpu.VMEM((NBUF, CH, I), jnp.float32),
            pltpu.VMEM((1, I), jnp.float32),
            pltpu.SemaphoreType.DMA((NBUF,)),
        ],
    )(x, W, b.reshape(1, I))


# speedup vs baseline: 1.0017x; 1.0017x over previous
import jax
import jax.numpy as jnp
from jax.experimental import pallas as pl
from jax.experimental.pallas import tpu as pltpu

B = 4096
I = 4096
O = 4096
CH = 128
NW = O // CH
NX = B // CH
NBUF = 24


def _body(x_hbm, w_hbm, b_ref, o_ref, buf, ws_ref, sems):
    def copy(i):
        if i < NW:
            src = w_hbm.at[pl.ds(i * CH, CH), :]
        else:
            src = x_hbm.at[pl.ds((i - NW) * CH, CH), :]
        return pltpu.make_async_copy(src, buf.at[i % NBUF], sems.at[i % NBUF])

    for i in range(NBUF):
        copy(i).start()

    bsum = jnp.sum(b_ref[...])

    for i in range(NW + NX):
        copy(i).wait()
        data = buf[i % NBUF]
        if i == 0:
            ws_ref[...] = jnp.sum(data, axis=0, keepdims=True)
        elif i < NW:
            ws_ref[...] += jnp.sum(data, axis=0, keepdims=True)
        else:
            part = jnp.sum(data * ws_ref[...], axis=1, keepdims=True)
            o_ref[pl.ds((i - NW) * CH, CH), :] = part + bsum
        if i + NBUF < NW + NX:
            copy(i + NBUF).start()


def kernel(x, W, b):
    return pl.pallas_call(
        _body,
        in_specs=[
            pl.BlockSpec(memory_space=pltpu.MemorySpace.HBM),
            pl.BlockSpec(memory_space=pltpu.MemorySpace.HBM),
            pl.BlockSpec((1, I), lambda: (0, 0)),
        ],
        out_specs=pl.BlockSpec((B, 1), lambda: (0, 0)),
        out_shape=jax.ShapeDtypeStruct((B, 1), jnp.float32),
        scratch_shapes=[
            pltpu.VMEM((NBUF, CH, I), jnp.float32),
            pltpu.VMEM((1, I), jnp.float32),
            pltpu.SemaphoreType.DMA((NBUF,)),
        ],
    )(x, W, b.reshape(1, I))


# speedup vs baseline: 1.0129x; 1.0111x over previous
import jax
import jax.numpy as jnp
from jax.experimental import pallas as pl
from jax.experimental.pallas import tpu as pltpu

B = 4096
I = 4096
O = 4096
CH = 128
NW = O // CH
NX = B // CH
NBUF = 16


def _body(x_hbm, w_hbm, b_ref, o_ref, buf, ws_ref, sems):
    def copy(i):
        if i < NW:
            src = w_hbm.at[pl.ds(i * CH, CH), :]
        else:
            src = x_hbm.at[pl.ds((i - NW) * CH, CH), :]
        return pltpu.make_async_copy(src, buf.at[i % NBUF], sems.at[i % NBUF])

    for i in range(NBUF):
        copy(i).start()

    bsum = jnp.sum(b_ref[...])

    for i in range(NW + NX):
        copy(i).wait()
        data = buf[i % NBUF]
        if i == 0:
            ws_ref[...] = jnp.sum(data, axis=0, keepdims=True)
        elif i < NW:
            ws_ref[...] += jnp.sum(data, axis=0, keepdims=True)
        else:
            part = jnp.sum(data * ws_ref[...], axis=1, keepdims=True)
            o_ref[pl.ds((i - NW) * CH, CH), :] = part + bsum
        if i + NBUF < NW + NX:
            copy(i + NBUF).start()


def kernel(x, W, b):
    return pl.pallas_call(
        _body,
        in_specs=[
            pl.BlockSpec(memory_space=pltpu.MemorySpace.HBM),
            pl.BlockSpec(memory_space=pltpu.MemorySpace.HBM),
            pl.BlockSpec((1, I), lambda: (0, 0)),
        ],
        out_specs=pl.BlockSpec((B, 1), lambda: (0, 0)),
        out_shape=jax.ShapeDtypeStruct((B, 1), jnp.float32),
        scratch_shapes=[
            pltpu.VMEM((NBUF, CH, I), jnp.float32),
            pltpu.VMEM((1, I), jnp.float32),
            pltpu.SemaphoreType.DMA((NBUF,)),
        ],
    )(x, W, b.reshape(1, I))
